# baseline (speedup 1.0000x reference)
import functools

import jax
import jax.numpy as jnp
from jax import lax
from jax.experimental import pallas as pl
from jax.experimental.pallas import tpu as pltpu
from jax.experimental.pallas import tpu_sc as plsc

_NC = 2
_NS = 16
_NW = _NC * _NS
_L = 16

_D = 128
_HALF = 64
_NJ = 32
_CH = 128


def _body(rows_per_w, ids_hbm, chain_hbm, depth_hbm, jtc_hbm, jtd_hbm,
          out_hbm, jtc_v, jtd_v, ce_v, de_v, fused_v, fused_sh, ids_v,
          rows0, rows1, gsem0, gsem1, ssem0, ssem1):
    cid = lax.axis_index("c")
    sid = lax.axis_index("s")
    wid = sid * _NC + cid

    pltpu.sync_copy(jtc_hbm, jtc_v)
    pltpu.sync_copy(jtd_hbm, jtd_v)
    pltpu.async_copy(chain_hbm.at[jtc_v], ce_v, gsem0).wait()
    pltpu.async_copy(depth_hbm.at[jtd_v], de_v, gsem0).wait()
    for j in range(_NJ):
        for v in range(_HALF // _L):
            fused_v[j, pl.ds(v * _L, _L)] = ce_v[j, pl.ds(v * _L, _L)]
            fused_v[j, pl.ds(_HALF + v * _L, _L)] = (
                de_v[j, pl.ds(v * _L, _L)])
    pltpu.sync_copy(fused_v, fused_sh.at[sid])

    rbase = pl.multiple_of(wid * rows_per_w, 8)
    pltpu.async_copy(ids_hbm.at[pl.ds(rbase, rows_per_w), :], ids_v,
                     gsem1).wait()
    plsc.subcore_barrier()

    slots = ((rows0, gsem0, ssem0), (rows1, gsem1, ssem1))

    def chunk_pair(k2, carry):
        for sl, (rows, gsem, ssem) in enumerate(slots):
            k = (k2 * 2 + sl) * 2

            @pl.when(k2 > 0)
            def _():
                pltpu.make_async_copy(
                    rows, out_hbm.at[pl.ds(0, 2 * _CH), :], ssem).wait()

            cp0 = pltpu.async_copy(
                fused_sh.at[sid].at[ids_v.at[k]], rows.at[pl.ds(0, _CH), :], gsem)
            cp1 = pltpu.async_copy(
                fused_sh.at[sid].at[ids_v.at[k + 1]], rows.at[pl.ds(_CH, _CH), :],
                gsem)
            cp0.wait()
            cp1.wait()
            off = pl.multiple_of((rbase + k) * _CH, _CH)
            pltpu.async_copy(rows, out_hbm.at[pl.ds(off, 2 * _CH), :], ssem)
        return carry

    lax.fori_loop(0, rows_per_w // 4, chunk_pair, 0)
    for rows, _, ssem in slots:
        pltpu.make_async_copy(rows, out_hbm.at[pl.ds(0, 2 * _CH), :],
                              ssem).wait()


@jax.jit
def _sc_encode(ids2d, chain_pad, depth_pad, jtc_pad, jtd_pad):
    n_rows = ids2d.shape[0]
    assert n_rows % (_NW * 4) == 0
    rows_per_w = n_rows // _NW

    mesh = plsc.VectorSubcoreMesh(core_axis_name="c", subcore_axis_name="s",
                                  num_cores=_NC, num_subcores=_NS)
    lookup = pl.kernel(
        functools.partial(_body, rows_per_w),
        out_type=jax.ShapeDtypeStruct((n_rows * _CH, _D), jnp.float32),
        mesh=mesh,
        compiler_params=pltpu.CompilerParams(needs_layout_passes=False),
        scratch_types=[
            pltpu.VMEM((_NJ,), jnp.int32),
            pltpu.VMEM((_NJ,), jnp.int32),
            pltpu.VMEM((_NJ, _D), jnp.float32),
            pltpu.VMEM((_NJ, _D), jnp.float32),
            pltpu.VMEM((_NJ, _D), jnp.float32),
            pltpu.VMEM_SHARED((_NS, _NJ, _D), jnp.float32),
            pltpu.VMEM((rows_per_w, _CH), jnp.int32),
            pltpu.VMEM((2 * _CH, _D), jnp.float32),
            pltpu.VMEM((2 * _CH, _D), jnp.float32),
            pltpu.SemaphoreType.DMA,
            pltpu.SemaphoreType.DMA,
            pltpu.SemaphoreType.DMA,
            pltpu.SemaphoreType.DMA,
        ],
    )
    return lookup(ids2d, chain_pad, depth_pad, jtc_pad, jtd_pad)


def kernel(joint_ids, chain_emb_weight, depth_emb_weight, joint_to_chain,
           joint_to_depth):
    b, t = joint_ids.shape
    ids2d = joint_ids.reshape(-1, _CH).astype(jnp.int32)
    jtc_pad = jnp.pad(joint_to_chain.astype(jnp.int32), (0, 10))
    jtd_pad = jnp.pad(joint_to_depth.astype(jnp.int32), (0, 10))
    ce_pad = jnp.pad(chain_emb_weight,
                     ((0, 8 - chain_emb_weight.shape[0]), (0, _D - _HALF)))
    de_pad = jnp.pad(depth_emb_weight,
                     ((0, 8 - depth_emb_weight.shape[0]), (0, _D - _HALF)))
    out = _sc_encode(ids2d, ce_pad, de_pad, jtc_pad, jtd_pad)
    return out.reshape(b, t, _D)


# speedup vs baseline: 1.1015x; 1.1015x over previous
import functools

import jax
import jax.numpy as jnp
from jax import lax
from jax.experimental import pallas as pl
from jax.experimental.pallas import tpu as pltpu
from jax.experimental.pallas import tpu_sc as plsc

_NC = 2
_NS = 16
_NW = _NC * _NS
_L = 16

_D = 128
_HALF = 64
_NJ = 32
_CH = 128


def _body(rows_per_w, ids_hbm, chain_hbm, depth_hbm, jtc_hbm, jtd_hbm,
          out_hbm, jtc_v, jtd_v, ce_v, de_v, fused_v, fused_sh, ids_v,
          rows0, rows1, gsem0, gsem1, ssem0, ssem1):
    cid = lax.axis_index("c")
    sid = lax.axis_index("s")
    wid = sid * _NC + cid

    @pl.when(sid == 0)
    def _():
        pltpu.sync_copy(jtc_hbm, jtc_v)
        pltpu.sync_copy(jtd_hbm, jtd_v)
        pltpu.async_copy(chain_hbm.at[jtc_v], ce_v, gsem0).wait()
        pltpu.async_copy(depth_hbm.at[jtd_v], de_v, gsem0).wait()
        for j in range(_NJ):
            for v in range(_HALF // _L):
                fused_v[j, pl.ds(v * _L, _L)] = ce_v[j, pl.ds(v * _L, _L)]
                fused_v[j, pl.ds(_HALF + v * _L, _L)] = (
                    de_v[j, pl.ds(v * _L, _L)])
        pltpu.sync_copy(fused_v, fused_sh)

    rbase = pl.multiple_of(wid * rows_per_w, 8)
    pltpu.async_copy(ids_hbm.at[pl.ds(rbase, rows_per_w), :], ids_v,
                     gsem1).wait()
    plsc.subcore_barrier()

    slots = ((rows0, gsem0, ssem0), (rows1, gsem1, ssem1))

    def chunk_pair(k2, carry):
        for sl, (rows, gsem, ssem) in enumerate(slots):
            k = (k2 * 2 + sl) * 2

            @pl.when(k2 > 0)
            def _():
                pltpu.make_async_copy(
                    rows, out_hbm.at[pl.ds(0, 2 * _CH), :], ssem).wait()

            cp0 = pltpu.async_copy(
                fused_sh.at[ids_v.at[k]], rows.at[pl.ds(0, _CH), :], gsem)
            cp1 = pltpu.async_copy(
                fused_sh.at[ids_v.at[k + 1]], rows.at[pl.ds(_CH, _CH), :],
                gsem)
            cp0.wait()
            cp1.wait()
            off = pl.multiple_of((rbase + k) * _CH, _CH)
            pltpu.async_copy(rows, out_hbm.at[pl.ds(off, 2 * _CH), :], ssem)
        return carry

    lax.fori_loop(0, rows_per_w // 4, chunk_pair, 0)
    for rows, _, ssem in slots:
        pltpu.make_async_copy(rows, out_hbm.at[pl.ds(0, 2 * _CH), :],
                              ssem).wait()


@jax.jit
def _sc_encode(ids2d, chain_pad, depth_pad, jtc_pad, jtd_pad):
    n_rows = ids2d.shape[0]
    assert n_rows % (_NW * 4) == 0
    rows_per_w = n_rows // _NW

    mesh = plsc.VectorSubcoreMesh(core_axis_name="c", subcore_axis_name="s",
                                  num_cores=_NC, num_subcores=_NS)
    lookup = pl.kernel(
        functools.partial(_body, rows_per_w),
        out_type=jax.ShapeDtypeStruct((n_rows * _CH, _D), jnp.float32),
        mesh=mesh,
        compiler_params=pltpu.CompilerParams(needs_layout_passes=False),
        scratch_types=[
            pltpu.VMEM((_NJ,), jnp.int32),
            pltpu.VMEM((_NJ,), jnp.int32),
            pltpu.VMEM((_NJ, _D), jnp.float32),
            pltpu.VMEM((_NJ, _D), jnp.float32),
            pltpu.VMEM((_NJ, _D), jnp.float32),
            pltpu.VMEM_SHARED((_NJ, _D), jnp.float32),
            pltpu.VMEM((rows_per_w, _CH), jnp.int32),
            pltpu.VMEM((2 * _CH, _D), jnp.float32),
            pltpu.VMEM((2 * _CH, _D), jnp.float32),
            pltpu.SemaphoreType.DMA,
            pltpu.SemaphoreType.DMA,
            pltpu.SemaphoreType.DMA,
            pltpu.SemaphoreType.DMA,
        ],
    )
    return lookup(ids2d, chain_pad, depth_pad, jtc_pad, jtd_pad)


def kernel(joint_ids, chain_emb_weight, depth_emb_weight, joint_to_chain,
           joint_to_depth):
    b, t = joint_ids.shape
    ids2d = joint_ids.reshape(-1, _CH).astype(jnp.int32)
    jtc_pad = jnp.pad(joint_to_chain.astype(jnp.int32), (0, 10))
    jtd_pad = jnp.pad(joint_to_depth.astype(jnp.int32), (0, 10))
    ce_pad = jnp.pad(chain_emb_weight,
                     ((0, 8 - chain_emb_weight.shape[0]), (0, _D - _HALF)))
    de_pad = jnp.pad(depth_emb_weight,
                     ((0, 8 - depth_emb_weight.shape[0]), (0, _D - _HALF)))
    out = _sc_encode(ids2d, ce_pad, de_pad, jtc_pad, jtd_pad)
    return out.reshape(b, t, _D)
